# baseline (speedup 1.0000x reference)
import functools

import jax
import jax.numpy as jnp
from jax import lax
from jax.experimental import pallas as pl
from jax.experimental.pallas import tpu as pltpu
from jax.experimental.pallas import tpu_sc as plsc

_TOPK = 30
_QBLK = 256
_NC, _NS, _L = 2, 16, 16
_NW = _NC * _NS
_CAP = 128
_GRP = 4


def _tc_sim_block(q_ref, mk_ref, sim_ref, m0_ref, tbar_ref, *, n_valid):
    q = q_ref[...]
    mk = mk_ref[...]
    sim = 2.0 * lax.dot_general(q, mk, (((1,), (1,)), ((), ())),
                                preferred_element_type=jnp.float32)
    ones = jnp.ones((1, mk.shape[1]), jnp.float32)
    ksq = lax.dot_general(ones, mk * mk, (((1,), (1,)), ((), ())),
                          precision=lax.Precision.HIGHEST,
                          preferred_element_type=jnp.float32)
    sim = sim - ksq
    mp = mk.shape[0]
    col = lax.broadcasted_iota(jnp.int32, (1, mp), 1)
    neg = jnp.float32(-jnp.inf)
    sim = jnp.where(col < n_valid, sim, neg)
    sim_ref[...] = sim
    m0_ref[...] = jnp.max(sim, axis=1)
    cm = sim[:, 0:128]
    for j in range(1, mp // 128):
        cm = jnp.maximum(cm, sim[:, j * 128:(j + 1) * 128])

    def body(_, m):
        return jnp.max(jnp.where(cm < m, cm, neg), axis=1, keepdims=True)

    tb = lax.fori_loop(0, _TOPK - 1, body, jnp.max(cm, axis=1, keepdims=True))
    tbar_ref[...] = tb[:, 0]


def _sc_body(sim_ref, tbar_ref, t30_ref, row_v, cand_v, tbar_v, t30_v, sem,
             *, qpw, mp):
    wid = lax.axis_index("s") * _NC + lax.axis_index("c")
    qbase = wid * qpw
    pltpu.sync_copy(tbar_ref.at[pl.ds(qbase, qpw)], tbar_v.at[pl.ds(0, qpw)])

    iota = lax.iota(jnp.int32, _L)
    neg = jnp.float32(-jnp.inf)
    negs = jnp.full((_L,), neg, jnp.float32)
    zeros_i = jnp.zeros((_L,), jnp.int32)

    def compress_row(bq, theta):
        th = jnp.full((_L,), theta, jnp.float32)

        def body(i, off_v):
            x = row_v[bq, pl.ds(i * _L, _L)]
            m = x >= th
            cs = plsc.cumsum(m.astype(jnp.int32))
            pos = jnp.where(m, off_v + cs - 1, 0)
            plsc.store_scatter(cand_v, [pos], x, mask=m)
            return off_v + plsc.all_reduce_population_count(m)

        off_v = plsc.parallel_loop(0, mp // _L, unroll=16, carry=zeros_i)(body)
        return jnp.max(off_v)

    def tighten(c):
        csp = jnp.full((_L,), c, jnp.int32)
        nch = (c + _L - 1) // _L

        def body_a(i, carry):
            m1, m2 = carry
            valid = (i * _L + iota) < csp
            x = jnp.where(valid, cand_v[pl.ds(i * _L, _L)], neg)
            return jnp.maximum(m1, x), jnp.maximum(m2, jnp.minimum(m1, x))

        m1, m2 = lax.fori_loop(0, nch, body_a, (negs, negs))
        theta = jnp.min(m2)
        th = jnp.full((_L,), theta, jnp.float32)

        def body_b(i, carry):
            cge_v, cgt_v = carry
            valid = (i * _L + iota) < csp
            x = jnp.where(valid, cand_v[pl.ds(i * _L, _L)], neg)
            cge_v = cge_v + plsc.all_reduce_population_count(x >= th)
            cgt_v = cgt_v + plsc.all_reduce_population_count(x > th)
            return cge_v, cgt_v

        cge_v, cgt_v = lax.fori_loop(0, nch, body_b, (zeros_i, zeros_i))
        cge = jnp.max(cge_v)
        cgt = jnp.max(cgt_v)
        done = cgt < _TOPK
        strict = cge >= c

        def body_c(i, off_v):
            valid = (i * _L + iota) < csp
            x = jnp.where(valid, cand_v[pl.ds(i * _L, _L)], neg)
            m = jnp.where(strict, x > th, x >= th)
            cs = plsc.cumsum(m.astype(jnp.int32))
            pos = jnp.where(m, off_v + cs - 1, 0)
            plsc.store_scatter(cand_v, [pos], x, mask=m)
            return off_v + plsc.all_reduce_population_count(m)

        off_v = lax.fori_loop(0, nch, body_c, zeros_i)
        return jnp.max(off_v), done, theta

    ngrp = qpw // _GRP

    def gloop(g, carry):
        b = lax.rem(g, 2)
        pltpu.make_async_copy(
            sim_ref.at[pl.ds(qbase + g * _GRP, _GRP)],
            row_v.at[pl.ds(b * _GRP, _GRP)], sem.at[b],
        ).wait()
        gn = jnp.minimum(g + 1, ngrp - 1)
        bn = lax.rem(g + 1, 2)

        @pl.when(g + 1 < ngrp)
        def _():
            pltpu.make_async_copy(
                sim_ref.at[pl.ds(qbase + gn * _GRP, _GRP)],
                row_v.at[pl.ds(bn * _GRP, _GRP)], sem.at[bn],
            ).start()

        for qj in range(_GRP):
            qi = g * _GRP + qj
            _one_query(qi, b * _GRP + qj)
        return carry

    def _one_query(qi, bq):
        c0 = compress_row(bq, tbar_v[pl.ds(qi, _L)][0])

        def w_cond(s):
            c, done, _ = s
            return jnp.logical_and(c > _CAP, jnp.logical_not(done))

        def w_body(s):
            c, _, _ = s
            return tighten(c)

        c, done, t_tie = lax.while_loop(
            w_cond, w_body, (c0, jnp.bool_(False), jnp.float32(0)))

        def sort_path():
            csp = jnp.full((_L,), c, jnp.int32)

            def merge(j, hh):
                h1, h2 = hh
                valid = (j * _L + iota) < csp
                x = jnp.where(valid, cand_v[pl.ds(j * _L, _L)], neg)
                sj, _ = plsc.sort_key_val(x, x, descending=True)
                xup = jnp.maximum(h2, jnp.flip(sj))
                xs, _ = plsc.sort_key_val(xup, xup, descending=True)
                ru = jnp.flip(xs)
                u = jnp.maximum(h1, ru)
                v = jnp.minimum(h1, ru)
                h1, _ = plsc.sort_key_val(u, u, descending=True)
                h2, _ = plsc.sort_key_val(v, v, descending=True)
                return h1, h2

            hh = merge(0, (negs, negs))
            hh = merge(1, hh)
            for j in range(2, _CAP // _L):
                hh = lax.cond(j * _L < c, lambda t, jj=j: merge(jj, t),
                              lambda t: t, hh)
            return hh[1][_TOPK - _L - 1]

        t30 = lax.cond(done, lambda: t_tie, sort_path)
        plsc.store_scatter(t30_v, [jnp.full((_L,), qi, jnp.int32)],
                           jnp.full((_L,), t30, jnp.float32), mask=iota == 0)

    pltpu.make_async_copy(
        sim_ref.at[pl.ds(qbase, _GRP)], row_v.at[pl.ds(0, _GRP)], sem.at[0],
    ).start()
    lax.fori_loop(0, ngrp, gloop, 0)
    pltpu.sync_copy(t30_v.at[pl.ds(0, qpw)], t30_ref.at[pl.ds(qbase, qpw)])


def _sc_select(sim, tbar, qn, mp):
    qpw = qn // _NW
    mesh = plsc.VectorSubcoreMesh(core_axis_name="c", subcore_axis_name="s")
    return pl.kernel(
        functools.partial(_sc_body, qpw=qpw, mp=mp),
        out_type=jax.ShapeDtypeStruct((qn,), jnp.float32),
        mesh=mesh,
        compiler_params=pltpu.CompilerParams(needs_layout_passes=False),
        scratch_types=[
            pltpu.VMEM((2 * _GRP, mp), jnp.float32),
            pltpu.VMEM((mp,), jnp.float32),
            pltpu.VMEM((qpw + _L,), jnp.float32),
            pltpu.VMEM((qpw,), jnp.float32),
            pltpu.SemaphoreType.DMA((2,)),
        ],
    )(sim, tbar)


def _tc_readout_block(sim_ref, m0_ref, t30_ref, mv_ref, o_ref):
    sim = sim_ref[...]
    m0 = m0_ref[...][:, None]
    t30 = t30_ref[...][:, None]
    w = jnp.where(sim >= t30, jnp.exp(sim - m0), 0.0)
    z = jnp.sum(w, axis=1, keepdims=True)
    n = mv_ref.shape[0]
    r = lax.dot_general(w[:, :n].astype(jnp.bfloat16), mv_ref[...],
                        (((1,), (0,)), ((), ())),
                        preferred_element_type=jnp.float32)
    o_ref[...] = r / z


_NSPLIT = 4


def _pipeline_slice(query, mk, mv, n_valid, mp):
    qn, d = query.shape
    nv, cv = mv.shape
    qb = _QBLK if qn % _QBLK == 0 else qn
    nblk = qn // qb
    sim, m0, tbar = pl.pallas_call(
        functools.partial(_tc_sim_block, n_valid=n_valid),
        grid=(nblk,),
        in_specs=[
            pl.BlockSpec((qb, d), lambda i: (i, 0)),
            pl.BlockSpec((mp, d), lambda i: (0, 0)),
        ],
        out_specs=[
            pl.BlockSpec((qb, mp), lambda i: (i, 0)),
            pl.BlockSpec((qb,), lambda i: (i,)),
            pl.BlockSpec((qb,), lambda i: (i,)),
        ],
        out_shape=[
            jax.ShapeDtypeStruct((qn, mp), jnp.float32),
            jax.ShapeDtypeStruct((qn,), jnp.float32),
            jax.ShapeDtypeStruct((qn,), jnp.float32),
        ],
    )(query, mk)
    t30 = _sc_select(sim, tbar, qn, mp)
    return pl.pallas_call(
        _tc_readout_block,
        grid=(nblk,),
        in_specs=[
            pl.BlockSpec((qb, mp), lambda i: (i, 0)),
            pl.BlockSpec((qb,), lambda i: (i,)),
            pl.BlockSpec((qb,), lambda i: (i,)),
            pl.BlockSpec((nv, cv), lambda i: (0, 0)),
        ],
        out_specs=pl.BlockSpec((qb, cv), lambda i: (i, 0)),
        out_shape=jax.ShapeDtypeStruct((qn, cv), jnp.float32),
    )(sim, m0, t30, mv)


def kernel(query, mem_key, mem_value, top_k):
    qn, d = query.shape
    n, cv = mem_value.shape
    mp = ((n + 1023) // 1024) * 1024
    mk = jnp.pad(mem_key, ((0, mp - n), (0, 0)))
    mv = mem_value.astype(jnp.bfloat16)
    ns = _NSPLIT if qn % (_NSPLIT * _NW * _GRP) == 0 else 1
    qs = qn // ns
    outs = [
        _pipeline_slice(query[i * qs:(i + 1) * qs], mk, mv, n, mp)
        for i in range(ns)
    ]
    return outs[0] if ns == 1 else jnp.concatenate(outs, axis=0)


# speedup vs baseline: 1.1372x; 1.1372x over previous
import functools

import jax
import jax.numpy as jnp
from jax import lax
from jax.experimental import pallas as pl
from jax.experimental.pallas import tpu as pltpu
from jax.experimental.pallas import tpu_sc as plsc

_TOPK = 30
_QBLK = 256
_NC, _NS, _L = 2, 16, 16
_NW = _NC * _NS
_CAP = 128
_GRP = 4


def _tc_ksq_block(mk_ref, ksq_ref):
    mk = mk_ref[...]
    ones = jnp.ones((1, mk.shape[1]), jnp.float32)
    ksq_ref[...] = lax.dot_general(ones, mk * mk, (((1,), (1,)), ((), ())),
                                   precision=lax.Precision.HIGHEST,
                                   preferred_element_type=jnp.float32)


def _tc_sim_block(q_ref, mk_ref, ksq_ref, sim_ref, m0_ref, tbar_ref, *, n_valid):
    q = q_ref[...]
    mk = mk_ref[...]
    sim = 2.0 * lax.dot_general(q, mk, (((1,), (1,)), ((), ())),
                                preferred_element_type=jnp.float32)
    sim = sim - ksq_ref[...]
    mp = mk.shape[0]
    col = lax.broadcasted_iota(jnp.int32, (1, mp), 1)
    neg = jnp.float32(-jnp.inf)
    sim = jnp.where(col < n_valid, sim, neg)
    sim_ref[...] = sim
    m0_ref[...] = jnp.max(sim, axis=1)
    cm = sim[:, 0:128]
    for j in range(1, mp // 128):
        cm = jnp.maximum(cm, sim[:, j * 128:(j + 1) * 128])

    def body(_, m):
        return jnp.max(jnp.where(cm < m, cm, neg), axis=1, keepdims=True)

    tb = lax.fori_loop(0, _TOPK - 1, body, jnp.max(cm, axis=1, keepdims=True))
    tbar_ref[...] = tb[:, 0]


def _sc_body(sim_ref, tbar_ref, t30_ref, row_v, cand_v, tbar_v, t30_v, sem,
             *, qpw, mp):
    wid = lax.axis_index("s") * _NC + lax.axis_index("c")
    qbase = wid * qpw
    pltpu.sync_copy(tbar_ref.at[pl.ds(qbase, qpw)], tbar_v.at[pl.ds(0, qpw)])

    iota = lax.iota(jnp.int32, _L)
    neg = jnp.float32(-jnp.inf)
    negs = jnp.full((_L,), neg, jnp.float32)
    zeros_i = jnp.zeros((_L,), jnp.int32)

    def compress_row(bq, theta):
        th = jnp.full((_L,), theta, jnp.float32)

        def body(i, off_v):
            x = row_v[bq, pl.ds(i * _L, _L)]
            m = x >= th
            cs = plsc.cumsum(m.astype(jnp.int32))
            pos = jnp.where(m, off_v + cs - 1, 0)
            plsc.store_scatter(cand_v, [pos], x, mask=m)
            return off_v + plsc.all_reduce_population_count(m)

        off_v = plsc.parallel_loop(0, mp // _L, unroll=16, carry=zeros_i)(body)
        return jnp.max(off_v)

    def tighten(c):
        csp = jnp.full((_L,), c, jnp.int32)
        nch = (c + _L - 1) // _L

        def body_a(i, carry):
            m1, m2 = carry
            valid = (i * _L + iota) < csp
            x = jnp.where(valid, cand_v[pl.ds(i * _L, _L)], neg)
            return jnp.maximum(m1, x), jnp.maximum(m2, jnp.minimum(m1, x))

        m1, m2 = lax.fori_loop(0, nch, body_a, (negs, negs))
        theta = jnp.min(m2)
        th = jnp.full((_L,), theta, jnp.float32)

        def body_b(i, carry):
            cge_v, cgt_v = carry
            valid = (i * _L + iota) < csp
            x = jnp.where(valid, cand_v[pl.ds(i * _L, _L)], neg)
            cge_v = cge_v + plsc.all_reduce_population_count(x >= th)
            cgt_v = cgt_v + plsc.all_reduce_population_count(x > th)
            return cge_v, cgt_v

        cge_v, cgt_v = lax.fori_loop(0, nch, body_b, (zeros_i, zeros_i))
        cge = jnp.max(cge_v)
        cgt = jnp.max(cgt_v)
        done = cgt < _TOPK
        strict = cge >= c

        def body_c(i, off_v):
            valid = (i * _L + iota) < csp
            x = jnp.where(valid, cand_v[pl.ds(i * _L, _L)], neg)
            m = jnp.where(strict, x > th, x >= th)
            cs = plsc.cumsum(m.astype(jnp.int32))
            pos = jnp.where(m, off_v + cs - 1, 0)
            plsc.store_scatter(cand_v, [pos], x, mask=m)
            return off_v + plsc.all_reduce_population_count(m)

        off_v = lax.fori_loop(0, nch, body_c, zeros_i)
        return jnp.max(off_v), done, theta

    ngrp = qpw // _GRP

    def gloop(g, carry):
        b = lax.rem(g, 2)
        pltpu.make_async_copy(
            sim_ref.at[pl.ds(qbase + g * _GRP, _GRP)],
            row_v.at[pl.ds(b * _GRP, _GRP)], sem.at[b],
        ).wait()
        gn = jnp.minimum(g + 1, ngrp - 1)
        bn = lax.rem(g + 1, 2)

        @pl.when(g + 1 < ngrp)
        def _():
            pltpu.make_async_copy(
                sim_ref.at[pl.ds(qbase + gn * _GRP, _GRP)],
                row_v.at[pl.ds(bn * _GRP, _GRP)], sem.at[bn],
            ).start()

        for qj in range(_GRP):
            qi = g * _GRP + qj
            _one_query(qi, b * _GRP + qj)
        return carry

    def _one_query(qi, bq):
        c0 = compress_row(bq, tbar_v[pl.ds(qi, _L)][0])

        def w_cond(s):
            c, done, _ = s
            return jnp.logical_and(c > _CAP, jnp.logical_not(done))

        def w_body(s):
            c, _, _ = s
            return tighten(c)

        c, done, t_tie = lax.while_loop(
            w_cond, w_body, (c0, jnp.bool_(False), jnp.float32(0)))

        def sort_path():
            csp = jnp.full((_L,), c, jnp.int32)

            def merge(j, hh):
                h1, h2 = hh
                valid = (j * _L + iota) < csp
                x = jnp.where(valid, cand_v[pl.ds(j * _L, _L)], neg)
                sj, _ = plsc.sort_key_val(x, x, descending=True)
                xup = jnp.maximum(h2, jnp.flip(sj))
                xs, _ = plsc.sort_key_val(xup, xup, descending=True)
                ru = jnp.flip(xs)
                u = jnp.maximum(h1, ru)
                v = jnp.minimum(h1, ru)
                h1, _ = plsc.sort_key_val(u, u, descending=True)
                h2, _ = plsc.sort_key_val(v, v, descending=True)
                return h1, h2

            hh = merge(0, (negs, negs))
            hh = merge(1, hh)
            for j in range(2, _CAP // _L):
                hh = lax.cond(j * _L < c, lambda t, jj=j: merge(jj, t),
                              lambda t: t, hh)
            return hh[1][_TOPK - _L - 1]

        t30 = lax.cond(done, lambda: t_tie, sort_path)
        plsc.store_scatter(t30_v, [jnp.full((_L,), qi, jnp.int32)],
                           jnp.full((_L,), t30, jnp.float32), mask=iota == 0)

    pltpu.make_async_copy(
        sim_ref.at[pl.ds(qbase, _GRP)], row_v.at[pl.ds(0, _GRP)], sem.at[0],
    ).start()
    lax.fori_loop(0, ngrp, gloop, 0)
    pltpu.sync_copy(t30_v.at[pl.ds(0, qpw)], t30_ref.at[pl.ds(qbase, qpw)])


def _sc_select(sim, tbar, qn, mp):
    qpw = qn // _NW
    mesh = plsc.VectorSubcoreMesh(core_axis_name="c", subcore_axis_name="s")
    return pl.kernel(
        functools.partial(_sc_body, qpw=qpw, mp=mp),
        out_type=jax.ShapeDtypeStruct((qn,), jnp.float32),
        mesh=mesh,
        compiler_params=pltpu.CompilerParams(needs_layout_passes=False),
        scratch_types=[
            pltpu.VMEM((2 * _GRP, mp), jnp.float32),
            pltpu.VMEM((mp,), jnp.float32),
            pltpu.VMEM((qpw + _L,), jnp.float32),
            pltpu.VMEM((qpw,), jnp.float32),
            pltpu.SemaphoreType.DMA((2,)),
        ],
    )(sim, tbar)


def _tc_readout_block(sim_ref, m0_ref, t30_ref, mv_ref, o_ref):
    sim = sim_ref[...]
    m0 = m0_ref[...][:, None]
    t30 = t30_ref[...][:, None]
    w = jnp.where(sim >= t30, jnp.exp(sim - m0), 0.0)
    z = jnp.sum(w, axis=1, keepdims=True)
    n = mv_ref.shape[0]
    r = lax.dot_general(w[:, :n].astype(jnp.bfloat16), mv_ref[...],
                        (((1,), (0,)), ((), ())),
                        preferred_element_type=jnp.float32)
    o_ref[...] = r / z


_NSPLIT = 4


def _pipeline_slice(query, mk, ksq, mv, n_valid, mp):
    qn, d = query.shape
    nv, cv = mv.shape
    qb = _QBLK if qn % _QBLK == 0 else qn
    nblk = qn // qb
    sim, m0, tbar = pl.pallas_call(
        functools.partial(_tc_sim_block, n_valid=n_valid),
        grid=(nblk,),
        in_specs=[
            pl.BlockSpec((qb, d), lambda i: (i, 0)),
            pl.BlockSpec((mp, d), lambda i: (0, 0)),
            pl.BlockSpec((1, mp), lambda i: (0, 0)),
        ],
        out_specs=[
            pl.BlockSpec((qb, mp), lambda i: (i, 0)),
            pl.BlockSpec((qb,), lambda i: (i,)),
            pl.BlockSpec((qb,), lambda i: (i,)),
        ],
        out_shape=[
            jax.ShapeDtypeStruct((qn, mp), jnp.float32),
            jax.ShapeDtypeStruct((qn,), jnp.float32),
            jax.ShapeDtypeStruct((qn,), jnp.float32),
        ],
    )(query, mk, ksq)
    t30 = _sc_select(sim, tbar, qn, mp)
    return pl.pallas_call(
        _tc_readout_block,
        grid=(nblk,),
        in_specs=[
            pl.BlockSpec((qb, mp), lambda i: (i, 0)),
            pl.BlockSpec((qb,), lambda i: (i,)),
            pl.BlockSpec((qb,), lambda i: (i,)),
            pl.BlockSpec((nv, cv), lambda i: (0, 0)),
        ],
        out_specs=pl.BlockSpec((qb, cv), lambda i: (i, 0)),
        out_shape=jax.ShapeDtypeStruct((qn, cv), jnp.float32),
    )(sim, m0, t30, mv)


def kernel(query, mem_key, mem_value, top_k):
    qn, d = query.shape
    n, cv = mem_value.shape
    mp = ((n + 1023) // 1024) * 1024
    mk = jnp.pad(mem_key, ((0, mp - n), (0, 0)))
    mv = mem_value.astype(jnp.bfloat16)
    ksq = pl.pallas_call(
        _tc_ksq_block,
        out_shape=jax.ShapeDtypeStruct((1, mp), jnp.float32),
    )(mk)
    ns = _NSPLIT if qn % (_NSPLIT * _NW * _GRP) == 0 else 1
    qs = qn // ns
    outs = [
        _pipeline_slice(query[i * qs:(i + 1) * qs], mk, ksq, mv, n, mp)
        for i in range(ns)
    ]
    return outs[0] if ns == 1 else jnp.concatenate(outs, axis=0)
